# baseline (speedup 1.0000x reference)
import functools

import jax
import jax.numpy as jnp
from jax import lax
from jax.experimental import pallas as pl
from jax.experimental.pallas import tpu as pltpu
from jax.experimental.pallas import tpu_sc as plsc

_NC = 2
_NS = 16
_NW = _NC * _NS


def kernel(users, pos_items, neg_items, user_table, item_table):
    B = users.shape[0]
    D = user_table.shape[1]
    b_per_w = B // _NW

    u_idx = users.astype(jnp.int32)
    p_idx = pos_items.astype(jnp.int32)
    n_idx = neg_items.astype(jnp.int32)

    mesh = plsc.VectorSubcoreMesh(core_axis_name="c", subcore_axis_name="s")
    out_sds = jax.ShapeDtypeStruct((B, D), user_table.dtype)

    @functools.partial(
        pl.kernel,
        mesh=mesh,
        compiler_params=pltpu.CompilerParams(use_tc_tiling_on_sc=False),
        out_type=(out_sds, out_sds, out_sds),
        scratch_types=[
            pltpu.VMEM((b_per_w,), jnp.int32),
            pltpu.VMEM((b_per_w,), jnp.int32),
            pltpu.VMEM((b_per_w,), jnp.int32),
            pltpu.VMEM((b_per_w, D), jnp.float32),
            pltpu.VMEM((b_per_w, D), jnp.float32),
            pltpu.VMEM((b_per_w, D), jnp.float32),
            pltpu.SemaphoreType.DMA,
            pltpu.SemaphoreType.DMA,
            pltpu.SemaphoreType.DMA,
        ],
    )
    def gather_kernel(ut_hbm, ui_hbm, pi_hbm, ni_hbm, it_hbm,
                      ou_hbm, op_hbm, on_hbm,
                      ui_v, pi_v, ni_v, ur_v, pr_v, nr_v,
                      sem_u, sem_p, sem_n):
        wid = lax.axis_index("s") * _NC + lax.axis_index("c")
        base = wid * b_per_w

        triples = (
            (ut_hbm, ui_hbm, ou_hbm, ui_v, ur_v, sem_u),
            (it_hbm, pi_hbm, op_hbm, pi_v, pr_v, sem_p),
            (it_hbm, ni_hbm, on_hbm, ni_v, nr_v, sem_n),
        )
        copies = []
        for tbl, idx_hbm, _out, idx_v, rows_v, sem in triples:
            pltpu.sync_copy(idx_hbm.at[pl.ds(base, b_per_w)], idx_v)
            copies.append(pltpu.async_copy(tbl.at[idx_v], rows_v, sem))
        for (tbl, _idx, out_hbm, _iv, rows_v, sem), cp in zip(triples, copies):
            cp.wait()
            pltpu.sync_copy(rows_v, out_hbm.at[pl.ds(base, b_per_w)])

    return gather_kernel(user_table, u_idx, p_idx, n_idx, item_table)
